# baseline (speedup 1.0000x reference)
import functools
import math

import jax
import jax.numpy as jnp
import numpy as np
from jax import lax
from jax.experimental import pallas as pl
from jax.experimental.pallas import tpu as pltpu
from jax.experimental.pallas import tpu_sc as plsc

_NW = 32
_NC = 2
_CH = 128
_NB = 4


def _scale_body(scale, t_ref, o_ref):
    c16 = float(np.float32(np.float16(scale)))
    c_total = c16 * 2.0**112
    c_sub = c16 * 2.0**-24

    def scale_half(h):
        s = h & 0x8000
        mag = h & 0x7FFF
        f_norm = lax.bitcast_convert_type(
            lax.shift_left(mag, 13), jnp.float32
        ) * c_total
        f_sub = mag.astype(jnp.float32) * c_sub
        f = jnp.where(mag < 1024, f_sub, f_norm)
        u = lax.bitcast_convert_type(f, jnp.int32)
        is_sub = u < (113 << 23)
        vsub = f + 0.5
        osub = lax.bitcast_convert_type(vsub, jnp.int32) - (126 << 23)
        mant_odd = lax.shift_right_logical(u, 13) & 1
        un = u + (((15 - 127) << 23) + 0xFFF) + mant_odd
        onorm = lax.shift_right_logical(un, 13)
        return jnp.where(is_sub, osub, onorm) | s

    w = t_ref[...]
    lo = w & 0xFFFF
    hi = lax.shift_right_logical(w, 16)
    o_ref[...] = scale_half(lo) | lax.shift_left(scale_half(hi), 16)


@functools.lru_cache(maxsize=None)
def _build_gather(n_rows, emb_w):
    dtype = jnp.int32
    rpw = n_rows // _NW
    t = rpw // _CH
    mesh = plsc.VectorSubcoreMesh(core_axis_name="c", subcore_axis_name="s")

    @functools.partial(
        pl.kernel,
        mesh=mesh,
        out_type=jax.ShapeDtypeStruct((n_rows, emb_w), dtype),
        scratch_types=[
            pltpu.VMEM((t, _CH), jnp.int32),
            pltpu.VMEM((_NB, _CH, emb_w), dtype),
            pltpu.SemaphoreType.DMA((_NB,)),
            pltpu.SemaphoreType.DMA((_NB,)),
        ],
        compiler_params=pltpu.CompilerParams(use_tc_tiling_on_sc=False),
    )
    def gather(table_hbm, idx_hbm, out_hbm, idx_v, bufs, sem_g, sem_w):
        wid = lax.axis_index("s") * _NC + lax.axis_index("c")
        base = wid * rpw

        pltpu.sync_copy(idx_hbm.at[pl.ds(wid * t, t)], idx_v)

        def start_gather(j, b):
            pltpu.make_async_copy(
                table_hbm.at[idx_v.at[j]], bufs.at[b], sem_g.at[b]
            ).start()

        def wait_gather(b):
            pltpu.make_async_copy(
                table_hbm.at[pl.ds(0, _CH)], bufs.at[b], sem_g.at[b]
            ).wait()

        def start_write(j, b):
            pltpu.make_async_copy(
                bufs.at[b], out_hbm.at[pl.ds(base + j * _CH, _CH)], sem_w.at[b]
            ).start()

        def wait_write(b):
            pltpu.make_async_copy(
                bufs.at[b], out_hbm.at[pl.ds(base, _CH)], sem_w.at[b]
            ).wait()

        for b in range(_NB):
            start_gather(b, b)

        def outer(i, carry):
            j0 = i * _NB
            for b in range(_NB):
                j = j0 + b
                wait_gather(b)
                start_write(j, b)
                wait_write(b)

                @pl.when(j + _NB < t)
                def _():
                    start_gather(j + _NB, b)

            return carry

        lax.fori_loop(0, t // _NB, outer, 0)

    return gather


def kernel(x, table):
    vocab, emb_dim = table.shape
    batch, seq = x.shape
    n_rows = batch * seq

    scale = math.sqrt(emb_dim)
    emb_w = emb_dim // 2

    t32 = lax.bitcast_convert_type(
        table.reshape(vocab, emb_w, 2), jnp.int32
    )

    n_words = vocab * emb_w
    tblk = 2000
    scaled = pl.pallas_call(
        functools.partial(_scale_body, scale),
        out_shape=jax.ShapeDtypeStruct((n_words // 128, 128), jnp.int32),
        grid=(n_words // 128 // tblk,),
        in_specs=[pl.BlockSpec((tblk, 128), lambda i: (i, 0))],
        out_specs=pl.BlockSpec((tblk, 128), lambda i: (i, 0)),
    )(t32.reshape(n_words // 128, 128))

    idx = x.reshape(n_rows // _CH, _CH)
    gather = _build_gather(n_rows, emb_w)
    out32 = gather(scaled.reshape(vocab, emb_w), idx)
    out = lax.bitcast_convert_type(out32, table.dtype)
    return out.reshape(batch, seq, emb_dim)


# speedup vs baseline: 3.9146x; 3.9146x over previous
import functools
import math

import jax
import jax.numpy as jnp
import numpy as np
from jax import lax
from jax.experimental import pallas as pl
from jax.experimental.pallas import tpu as pltpu
from jax.experimental.pallas import tpu_sc as plsc

_NW = 32
_NC = 2
_CH = 128
_NB = 4


def _scale_words(w, scale):
    c16 = float(np.float32(np.float16(scale)))
    c_total = c16 * 2.0**112
    c_sub = c16 * 2.0**-24

    def scale_half(h):
        s = h & 0x8000
        mag = h & 0x7FFF
        f_norm = lax.bitcast_convert_type(
            lax.shift_left(mag, 13), jnp.float32
        ) * c_total
        f_sub = mag.astype(jnp.float32) * c_sub
        f = jnp.where(mag < 1024, f_sub, f_norm)
        u = lax.bitcast_convert_type(f, jnp.int32)
        is_sub = u < (113 << 23)
        vsub = f + 0.5
        osub = lax.bitcast_convert_type(vsub, jnp.int32) - (126 << 23)
        mant_odd = lax.shift_right_logical(u, 13) & 1
        un = u + (((15 - 127) << 23) + 0xFFF) + mant_odd
        onorm = lax.shift_right_logical(un, 13)
        return jnp.where(is_sub, osub, onorm) | s

    lo = scale_half(w & 0xFFFF)
    hi = scale_half(lax.shift_right_logical(w, 16))
    return lo, hi


def _scale_dup_body(scale, t_ref, o_ref):
    w = t_ref.bitcast(jnp.int32)[...]
    lo, hi = _scale_words(w, scale)
    o_ref[0] = lo | lax.shift_left(lo, 16)
    o_ref[1] = hi | lax.shift_left(hi, 16)


def _finish_body(w_ref, o_ref):
    o_ref.bitcast(jnp.int32)[...] = w_ref[...]


@functools.lru_cache(maxsize=None)
def _build_gather(n_rows, emb_w, half_vocab):
    rpw = n_rows // _NW
    t = rpw // _CH
    dw = 2 * emb_w
    mesh = plsc.VectorSubcoreMesh(core_axis_name="c", subcore_axis_name="s")

    @functools.partial(
        pl.kernel,
        mesh=mesh,
        out_type=jax.ShapeDtypeStruct((n_rows // 2, dw), jnp.int32),
        scratch_types=[
            pltpu.VMEM((t, _CH), jnp.int32),
            pltpu.VMEM((_NB, _CH, dw), jnp.int32),
            pltpu.VMEM((2, _CH // 2, dw), jnp.int32),
            pltpu.SemaphoreType.DMA((_NB,)),
            pltpu.SemaphoreType.DMA((2,)),
        ],
        compiler_params=pltpu.CompilerParams(use_tc_tiling_on_sc=True),
    )
    def gather(table_hbm, idx_hbm, out_hbm, idx_v, bufs, obufs, sem_g, sem_w):
        wid = lax.axis_index("s") * _NC + lax.axis_index("c")
        wbase = wid * (rpw // 2)

        pltpu.sync_copy(idx_hbm.at[pl.ds(wid * t, t)], idx_v)

        def remap(r, carry):
            for kk in range(_CH // 16):
                v = idx_v[r, pl.ds(16 * kk, 16)]
                idx_v[r, pl.ds(16 * kk, 16)] = (
                    (v & 1) * half_vocab + lax.shift_right_logical(v, 1)
                )
            return carry

        lax.fori_loop(0, t, remap, 0)

        def start_gather(j, b):
            pltpu.make_async_copy(
                table_hbm.at[idx_v.at[j]], bufs.at[b], sem_g.at[b]
            ).start()

        def wait_gather(b):
            pltpu.make_async_copy(
                table_hbm.at[pl.ds(0, _CH)], bufs.at[b], sem_g.at[b]
            ).wait()

        def start_write(j, ob):
            pltpu.make_async_copy(
                obufs.at[ob],
                out_hbm.at[pl.ds(wbase + j * (_CH // 2), _CH // 2)],
                sem_w.at[ob],
            ).start()

        def wait_write(ob):
            pltpu.make_async_copy(
                obufs.at[ob], out_hbm.at[pl.ds(wbase, _CH // 2)], sem_w.at[ob]
            ).wait()

        for b in range(_NB):
            start_gather(b, b)

        def compose(b, ob):
            def pair(i, carry):
                for kk in range(dw // 16):
                    ga = bufs[b, 2 * i, pl.ds(16 * kk, 16)]
                    gb = bufs[b, 2 * i + 1, pl.ds(16 * kk, 16)]
                    obufs[ob, i, pl.ds(16 * kk, 16)] = (ga & 0xFFFF) | (
                        gb & ~0xFFFF
                    )
                return carry

            lax.fori_loop(0, _CH // 2, pair, 0)

        def outer(jj, carry):
            for b in range(_NB):
                ob = b % 2
                j = jj * _NB + b
                wait_gather(b)

                @pl.when(j >= 2)
                def _():
                    wait_write(ob)

                compose(b, ob)
                start_write(j, ob)

                @pl.when(j + _NB < t)
                def _():
                    start_gather(j + _NB, b)

            return carry

        lax.fori_loop(0, t // _NB, outer, 0)
        for ob in range(2):
            wait_write(ob)

    return gather


def kernel(x, table):
    vocab, emb_dim = table.shape
    batch, seq = x.shape
    n_rows = batch * seq
    emb_w = emb_dim // 2
    scale = math.sqrt(emb_dim)

    tbits = lax.bitcast_convert_type(table, jnp.bfloat16)

    tblk = 4000
    p3 = pl.pallas_call(
        functools.partial(_scale_dup_body, scale),
        out_shape=jax.ShapeDtypeStruct((2, vocab // 2, 2 * emb_w), jnp.int32),
        grid=(vocab // tblk,),
        in_specs=[pl.BlockSpec((tblk, emb_dim), lambda i: (i, 0))],
        out_specs=pl.BlockSpec(
            (2, tblk // 2, 2 * emb_w), lambda i: (0, i, 0)
        ),
    )(tbits)

    idx = x.reshape(n_rows // _CH, _CH)
    gather = _build_gather(n_rows, emb_w, vocab // 2)
    out32 = gather(p3.reshape(vocab, 2 * emb_w), idx)

    fblk = 16
    out = pl.pallas_call(
        _finish_body,
        out_shape=jax.ShapeDtypeStruct((batch, seq, emb_dim), jnp.bfloat16),
        grid=(batch // fblk,),
        in_specs=[
            pl.BlockSpec((fblk, seq // 2, 2 * emb_w), lambda i: (i, 0, 0))
        ],
        out_specs=pl.BlockSpec((fblk, seq, emb_dim), lambda i: (i, 0, 0)),
    )(out32.reshape(batch, seq // 2, 2 * emb_w))
    return lax.bitcast_convert_type(out, table.dtype)


# speedup vs baseline: 5.3787x; 1.3740x over previous
import functools
import math

import jax
import jax.numpy as jnp
import numpy as np
from jax import lax
from jax.experimental import pallas as pl
from jax.experimental.pallas import tpu as pltpu
from jax.experimental.pallas import tpu_sc as plsc

_NW = 32
_NC = 2
_CH = 128
_NB = 4


def _scale_words(w, scale):
    c16 = float(np.float32(np.float16(scale)))
    c_total = c16 * 2.0**112
    c_sub = c16 * 2.0**-24

    def scale_half(h):
        s = h & 0x8000
        mag = h & 0x7FFF
        f_norm = lax.bitcast_convert_type(
            lax.shift_left(mag, 13), jnp.float32
        ) * c_total
        f_sub = mag.astype(jnp.float32) * c_sub
        f = jnp.where(mag < 1024, f_sub, f_norm)
        u = lax.bitcast_convert_type(f, jnp.int32)
        is_sub = u < (113 << 23)
        vsub = f + 0.5
        osub = lax.bitcast_convert_type(vsub, jnp.int32) - (126 << 23)
        mant_odd = lax.shift_right_logical(u, 13) & 1
        un = u + (((15 - 127) << 23) + 0xFFF) + mant_odd
        onorm = lax.shift_right_logical(un, 13)
        return jnp.where(is_sub, osub, onorm) | s

    lo = scale_half(w & 0xFFFF)
    hi = scale_half(lax.shift_right_logical(w, 16))
    return lo, hi


def _scale_dup_body(scale, t_ref, o_ref):
    w = t_ref.bitcast(jnp.int32)[...]
    lo, hi = _scale_words(w, scale)
    o_ref[0] = lo | lax.shift_left(lo, 16)
    o_ref[1] = hi | lax.shift_left(hi, 16)


def _finish_body(w_ref, o_ref):
    o_ref.bitcast(jnp.int32)[...] = w_ref[...]


@functools.lru_cache(maxsize=None)
def _build_gather(n_rows, emb_w, half_vocab):
    rpw = n_rows // _NW
    t = rpw // _CH
    dw = 2 * emb_w
    mesh = plsc.VectorSubcoreMesh(core_axis_name="c", subcore_axis_name="s")

    @functools.partial(
        pl.kernel,
        mesh=mesh,
        out_type=jax.ShapeDtypeStruct((n_rows // 2, dw), jnp.int32),
        scratch_types=[
            pltpu.VMEM((t, _CH), jnp.int32),
            pltpu.VMEM((_NB, _CH, dw), jnp.int32),
            pltpu.VMEM((2, _CH // 2, dw), jnp.int32),
            pltpu.SemaphoreType.DMA((_NB,)),
            pltpu.SemaphoreType.DMA((2,)),
        ],
        compiler_params=pltpu.CompilerParams(use_tc_tiling_on_sc=True),
    )
    def gather(table_hbm, idx_hbm, out_hbm, idx_v, bufs, obufs, sem_g, sem_w):
        wid = lax.axis_index("s") * _NC + lax.axis_index("c")
        wbase = wid * (rpw // 2)

        pltpu.sync_copy(idx_hbm.at[pl.ds(wid * t, t)], idx_v)

        def remap(r, carry):
            for kk in range(_CH // 16):
                v = idx_v[r, pl.ds(16 * kk, 16)]
                idx_v[r, pl.ds(16 * kk, 16)] = (
                    (v & 1) * half_vocab + lax.shift_right_logical(v, 1)
                )
            return carry

        lax.fori_loop(0, t, remap, 0)

        def start_gather(j, b):
            pltpu.make_async_copy(
                table_hbm.at[idx_v.at[j]], bufs.at[b], sem_g.at[b]
            ).start()

        def wait_gather(b):
            pltpu.make_async_copy(
                table_hbm.at[pl.ds(0, _CH)], bufs.at[b], sem_g.at[b]
            ).wait()

        def start_write(j, ob):
            pltpu.make_async_copy(
                obufs.at[ob],
                out_hbm.at[pl.ds(wbase + j * (_CH // 2), _CH // 2)],
                sem_w.at[ob],
            ).start()

        def wait_write(ob):
            pltpu.make_async_copy(
                obufs.at[ob], out_hbm.at[pl.ds(wbase, _CH // 2)], sem_w.at[ob]
            ).wait()

        for b in range(_NB):
            start_gather(b, b)

        def compose(b, ob):
            @functools.partial(
                plsc.parallel_loop, 0, _CH // 2, unroll=2
            )
            def pair(i):
                for kk in range(dw // 16):
                    ga = bufs[b, 2 * i, pl.ds(16 * kk, 16)]
                    gb = bufs[b, 2 * i + 1, pl.ds(16 * kk, 16)]
                    obufs[ob, i, pl.ds(16 * kk, 16)] = (ga & 0xFFFF) | (
                        gb & ~0xFFFF
                    )

        def outer(jj, carry):
            for b in range(_NB):
                ob = b % 2
                j = jj * _NB + b
                wait_gather(b)

                @pl.when(j >= 2)
                def _():
                    wait_write(ob)

                compose(b, ob)
                start_write(j, ob)

                @pl.when(j + _NB < t)
                def _():
                    start_gather(j + _NB, b)

            return carry

        lax.fori_loop(0, t // _NB, outer, 0)
        for ob in range(2):
            wait_write(ob)

    return gather


def kernel(x, table):
    vocab, emb_dim = table.shape
    batch, seq = x.shape
    n_rows = batch * seq
    emb_w = emb_dim // 2
    scale = math.sqrt(emb_dim)

    tbits = lax.bitcast_convert_type(table, jnp.bfloat16)

    tblk = 4000
    p3 = pl.pallas_call(
        functools.partial(_scale_dup_body, scale),
        out_shape=jax.ShapeDtypeStruct((2, vocab // 2, 2 * emb_w), jnp.int32),
        grid=(vocab // tblk,),
        in_specs=[pl.BlockSpec((tblk, emb_dim), lambda i: (i, 0))],
        out_specs=pl.BlockSpec(
            (2, tblk // 2, 2 * emb_w), lambda i: (0, i, 0)
        ),
    )(tbits)

    idx = x.reshape(n_rows // _CH, _CH)
    gather = _build_gather(n_rows, emb_w, vocab // 2)
    out32 = gather(p3.reshape(vocab, 2 * emb_w), idx)

    fblk = 16
    out = pl.pallas_call(
        _finish_body,
        out_shape=jax.ShapeDtypeStruct((batch, seq, emb_dim), jnp.bfloat16),
        grid=(batch // fblk,),
        in_specs=[
            pl.BlockSpec((fblk, seq // 2, 2 * emb_w), lambda i: (i, 0, 0))
        ],
        out_specs=pl.BlockSpec((fblk, seq, emb_dim), lambda i: (i, 0, 0)),
    )(out32.reshape(batch, seq // 2, 2 * emb_w))
    return lax.bitcast_convert_type(out, table.dtype)


# speedup vs baseline: 6.9754x; 1.2969x over previous
import functools
import math

import jax
import jax.numpy as jnp
import numpy as np
from jax import lax
from jax.experimental import pallas as pl
from jax.experimental.pallas import tpu as pltpu
from jax.experimental.pallas import tpu_sc as plsc

_NW = 32
_NC = 2
_CH = 128
_NB = 4


def _scale_words(w, scale):
    c16 = float(np.float32(np.float16(scale)))
    c_total = c16 * 2.0**112
    c_sub = c16 * 2.0**-24

    def scale_half(h):
        s = h & 0x8000
        mag = h & 0x7FFF
        f_norm = lax.bitcast_convert_type(
            lax.shift_left(mag, 13), jnp.float32
        ) * c_total
        f_sub = mag.astype(jnp.float32) * c_sub
        f = jnp.where(mag < 1024, f_sub, f_norm)
        u = lax.bitcast_convert_type(f, jnp.int32)
        is_sub = u < (113 << 23)
        vsub = f + 0.5
        osub = lax.bitcast_convert_type(vsub, jnp.int32) - (126 << 23)
        mant_odd = lax.shift_right_logical(u, 13) & 1
        un = u + (((15 - 127) << 23) + 0xFFF) + mant_odd
        onorm = lax.shift_right_logical(un, 13)
        return jnp.where(is_sub, osub, onorm) | s

    lo = scale_half(w & 0xFFFF)
    hi = scale_half(lax.shift_right_logical(w, 16))
    return lo, hi


def _scale_dup_body(scale, t_ref, o_ref):
    w = t_ref.bitcast(jnp.int32)[...]
    lo, hi = _scale_words(w, scale)
    o_ref[0] = lo | lax.shift_left(lo, 16)
    o_ref[1] = hi | lax.shift_left(hi, 16)


def _finish_body(w_ref, o_ref):
    o_ref.bitcast(jnp.int32)[...] = w_ref[...]


@functools.lru_cache(maxsize=None)
def _build_gather(n_rows, emb_w, half_vocab):
    rpw = n_rows // _NW
    t = rpw // _CH
    dw = 2 * emb_w
    mesh = plsc.VectorSubcoreMesh(core_axis_name="c", subcore_axis_name="s")

    @functools.partial(
        pl.kernel,
        mesh=mesh,
        out_type=jax.ShapeDtypeStruct((n_rows // 2, dw), jnp.int32),
        scratch_types=[
            pltpu.VMEM((t, _CH), jnp.int32),
            pltpu.VMEM((_NB, _CH, dw), jnp.int32),
            pltpu.VMEM((2, _CH // 2, dw), jnp.int32),
            pltpu.SemaphoreType.DMA((_NB,)),
            pltpu.SemaphoreType.DMA((2,)),
        ],
        compiler_params=pltpu.CompilerParams(use_tc_tiling_on_sc=True),
    )
    def gather(table_hbm, idx_hbm, out_hbm, idx_v, bufs, obufs, sem_g, sem_w):
        wid = lax.axis_index("s") * _NC + lax.axis_index("c")
        wbase = wid * (rpw // 2)

        pltpu.sync_copy(idx_hbm.at[pl.ds(wid * t, t)], idx_v)

        def remap(r, carry):
            for kk in range(_CH // 16):
                v = idx_v[r, pl.ds(16 * kk, 16)]
                idx_v[r, pl.ds(16 * kk, 16)] = (
                    (v & 1) * half_vocab + lax.shift_right_logical(v, 1)
                )
            return carry

        lax.fori_loop(0, t, remap, 0)

        def start_gather(j, b):
            pltpu.make_async_copy(
                table_hbm.at[idx_v.at[j]], bufs.at[b], sem_g.at[b]
            ).start()

        def wait_gather(b):
            pltpu.make_async_copy(
                table_hbm.at[pl.ds(0, _CH)], bufs.at[b], sem_g.at[b]
            ).wait()

        def start_write(j, ob):
            pltpu.make_async_copy(
                obufs.at[ob],
                out_hbm.at[pl.ds(wbase + j * (_CH // 2), _CH // 2)],
                sem_w.at[ob],
            ).start()

        def wait_write(ob):
            pltpu.make_async_copy(
                obufs.at[ob], out_hbm.at[pl.ds(wbase, _CH // 2)], sem_w.at[ob]
            ).wait()

        for b in range(_NB):
            start_gather(b, b)

        def compose(b, ob):
            @functools.partial(
                plsc.parallel_loop, 0, _CH // 2, unroll=2
            )
            def pair(i):
                for kk in range(dw // 16):
                    ga = bufs[b, 2 * i, pl.ds(16 * kk, 16)]
                    gb = bufs[b, 2 * i + 1, pl.ds(16 * kk, 16)]
                    obufs[ob, i, pl.ds(16 * kk, 16)] = (ga & 0xFFFF) | (
                        gb & ~0xFFFF
                    )

        def outer(jj, carry):
            for b in range(_NB):
                ob = b % 2
                j = jj * _NB + b
                wait_gather(b)

                @pl.when(j >= 2)
                def _():
                    wait_write(ob)

                compose(b, ob)
                start_write(j, ob)

                @pl.when(j + _NB < t)
                def _():
                    start_gather(j + _NB, b)

            return carry

        lax.fori_loop(0, t // _NB, outer, 0)
        for ob in range(2):
            wait_write(ob)

    return gather


def kernel(x, table):
    vocab, emb_dim = table.shape
    batch, seq = x.shape
    n_rows = batch * seq
    emb_w = emb_dim // 2
    scale = math.sqrt(emb_dim)

    tbits = lax.bitcast_convert_type(table, jnp.bfloat16)

    tblk = 4000
    p3 = pl.pallas_call(
        functools.partial(_scale_dup_body, scale),
        out_shape=jax.ShapeDtypeStruct((2, vocab // 2, 2 * emb_w), jnp.int32),
        grid=(vocab // tblk,),
        in_specs=[pl.BlockSpec((tblk, emb_dim), lambda i: (i, 0))],
        out_specs=pl.BlockSpec(
            (2, tblk // 2, 2 * emb_w), lambda i: (0, i, 0)
        ),
    )(tbits)

    idx = x.reshape(n_rows // _CH, _CH)
    gather = _build_gather(n_rows, emb_w, vocab // 2)
    out32 = gather(p3.reshape(vocab, 2 * emb_w), idx)

    fblk = 3200
    out = pl.pallas_call(
        _finish_body,
        out_shape=jax.ShapeDtypeStruct((n_rows, emb_dim), jnp.bfloat16),
        grid=(n_rows // fblk,),
        in_specs=[pl.BlockSpec((fblk // 2, 2 * emb_w), lambda i: (i, 0))],
        out_specs=pl.BlockSpec((fblk, emb_dim), lambda i: (i, 0)),
    )(out32)
    out = lax.bitcast_convert_type(out, table.dtype)
    return out.reshape(batch, seq, emb_dim)


# speedup vs baseline: 8.1038x; 1.1618x over previous
import functools
import math

import jax
import jax.numpy as jnp
import numpy as np
from jax import lax
from jax.experimental import pallas as pl
from jax.experimental.pallas import tpu as pltpu
from jax.experimental.pallas import tpu_sc as plsc

_NW = 32
_NC = 2
_CH = 128
_NB = 4


def _scale_words(w, scale):
    c16 = float(np.float32(np.float16(scale)))
    c_total = c16 * 2.0**112
    c_sub = c16 * 2.0**-24

    def scale_half(h):
        s = h & 0x8000
        mag = h & 0x7FFF
        f_norm = lax.bitcast_convert_type(
            lax.shift_left(mag, 13), jnp.float32
        ) * c_total
        f_sub = mag.astype(jnp.float32) * c_sub
        f = jnp.where(mag < 1024, f_sub, f_norm)
        u = lax.bitcast_convert_type(f, jnp.int32)
        is_sub = u < (113 << 23)
        vsub = f + 0.5
        osub = lax.bitcast_convert_type(vsub, jnp.int32) - (126 << 23)
        mant_odd = lax.shift_right_logical(u, 13) & 1
        un = u + (((15 - 127) << 23) + 0xFFF) + mant_odd
        onorm = lax.shift_right_logical(un, 13)
        return jnp.where(is_sub, osub, onorm) | s

    lo = scale_half(w & 0xFFFF)
    hi = scale_half(lax.shift_right_logical(w, 16))
    return lo, hi


def _scale_dup_body(scale, t_ref, o_ref):
    w = t_ref.bitcast(jnp.int32)[...]
    lo, hi = _scale_words(w, scale)
    o_ref[0] = lo | lax.shift_left(lo, 16)
    o_ref[1] = hi | lax.shift_left(hi, 16)


def _finish_body(w_ref, o_ref):
    o_ref.bitcast(jnp.int32)[...] = w_ref[...]


@functools.lru_cache(maxsize=None)
def _build_gather(n_rows, emb_w, half_vocab):
    rpw = n_rows // _NW
    t = rpw // _CH
    dw = 2 * emb_w
    mesh = plsc.VectorSubcoreMesh(core_axis_name="c", subcore_axis_name="s")

    @functools.partial(
        pl.kernel,
        mesh=mesh,
        out_type=jax.ShapeDtypeStruct((n_rows // 2, dw), jnp.int32),
        scratch_types=[
            pltpu.VMEM((t, _CH), jnp.int32),
            pltpu.VMEM((_NB, _CH, dw), jnp.int32),
            pltpu.VMEM((2, _CH // 2, dw), jnp.int32),
            pltpu.SemaphoreType.DMA((_NB,)),
            pltpu.SemaphoreType.DMA((2,)),
        ],
        compiler_params=pltpu.CompilerParams(use_tc_tiling_on_sc=True),
    )
    def gather(table_hbm, idx_hbm, out_hbm, idx_v, bufs, obufs, sem_g, sem_w):
        wid = lax.axis_index("s") * _NC + lax.axis_index("c")
        wbase = wid * (rpw // 2)

        pltpu.sync_copy(idx_hbm.at[pl.ds(wid * t, t)], idx_v)

        def remap(r, carry):
            for kk in range(_CH // 16):
                v = idx_v[r, pl.ds(16 * kk, 16)]
                idx_v[r, pl.ds(16 * kk, 16)] = (
                    (v & 1) * half_vocab + lax.shift_right_logical(v, 1)
                )
            return carry

        lax.fori_loop(0, t, remap, 0)

        def start_gather(j, b):
            pltpu.make_async_copy(
                table_hbm.at[idx_v.at[j]], bufs.at[b], sem_g.at[b]
            ).start()

        def wait_gather(b):
            pltpu.make_async_copy(
                table_hbm.at[pl.ds(0, _CH)], bufs.at[b], sem_g.at[b]
            ).wait()

        def start_write(j, ob):
            pltpu.make_async_copy(
                obufs.at[ob],
                out_hbm.at[pl.ds(wbase + j * (_CH // 2), _CH // 2)],
                sem_w.at[ob],
            ).start()

        def wait_write(ob):
            pltpu.make_async_copy(
                obufs.at[ob], out_hbm.at[pl.ds(wbase, _CH // 2)], sem_w.at[ob]
            ).wait()

        for b in range(_NB):
            start_gather(b, b)

        def compose(b, ob):
            @functools.partial(
                plsc.parallel_loop, 0, _CH // 2, unroll=4
            )
            def pair(i):
                for kk in range(dw // 16):
                    ga = bufs[b, 2 * i, pl.ds(16 * kk, 16)]
                    gb = bufs[b, 2 * i + 1, pl.ds(16 * kk, 16)]
                    obufs[ob, i, pl.ds(16 * kk, 16)] = (ga & 0xFFFF) | (
                        gb & ~0xFFFF
                    )

        def outer(jj, carry):
            for b in range(_NB):
                ob = b % 2
                j = jj * _NB + b
                wait_gather(b)

                @pl.when(j >= 2)
                def _():
                    wait_write(ob)

                compose(b, ob)
                start_write(j, ob)

                @pl.when(j + _NB < t)
                def _():
                    start_gather(j + _NB, b)

            return carry

        lax.fori_loop(0, t // _NB, outer, 0)
        for ob in range(2):
            wait_write(ob)

    return gather


def kernel(x, table):
    vocab, emb_dim = table.shape
    batch, seq = x.shape
    n_rows = batch * seq
    emb_w = emb_dim // 2
    scale = math.sqrt(emb_dim)

    tbits = lax.bitcast_convert_type(table, jnp.bfloat16)

    tblk = 4000
    p3 = pl.pallas_call(
        functools.partial(_scale_dup_body, scale),
        out_shape=jax.ShapeDtypeStruct((2, vocab // 2, 2 * emb_w), jnp.int32),
        grid=(vocab // tblk,),
        in_specs=[pl.BlockSpec((tblk, emb_dim), lambda i: (i, 0))],
        out_specs=pl.BlockSpec(
            (2, tblk // 2, 2 * emb_w), lambda i: (0, i, 0)
        ),
    )(tbits)

    idx = x.reshape(n_rows // _CH, _CH)
    gather = _build_gather(n_rows, emb_w, vocab // 2)
    out32 = gather(p3.reshape(vocab, 2 * emb_w), idx)

    fblk = 12800
    out = pl.pallas_call(
        _finish_body,
        out_shape=jax.ShapeDtypeStruct((n_rows, emb_dim), jnp.bfloat16),
        grid=(n_rows // fblk,),
        in_specs=[pl.BlockSpec((fblk // 2, 2 * emb_w), lambda i: (i, 0))],
        out_specs=pl.BlockSpec((fblk, emb_dim), lambda i: (i, 0)),
    )(out32)
    out = lax.bitcast_convert_type(out, table.dtype)
    return out.reshape(batch, seq, emb_dim)


# speedup vs baseline: 8.1522x; 1.0060x over previous
import functools
import math

import jax
import jax.numpy as jnp
import numpy as np
from jax import lax
from jax.experimental import pallas as pl
from jax.experimental.pallas import tpu as pltpu
from jax.experimental.pallas import tpu_sc as plsc

_NW = 32
_NC = 2
_CH = 128
_NB = 4


def _scale_words(w, scale):
    c16 = float(np.float32(np.float16(scale)))
    c_total = c16 * 2.0**112
    c_sub = c16 * 2.0**-24

    def scale_half(h):
        s = h & 0x8000
        mag = h & 0x7FFF
        f_norm = lax.bitcast_convert_type(
            lax.shift_left(mag, 13), jnp.float32
        ) * c_total
        f_sub = mag.astype(jnp.float32) * c_sub
        f = jnp.where(mag < 1024, f_sub, f_norm)
        u = lax.bitcast_convert_type(f, jnp.int32)
        is_sub = u < (113 << 23)
        vsub = f + 0.5
        osub = lax.bitcast_convert_type(vsub, jnp.int32) - (126 << 23)
        mant_odd = lax.shift_right_logical(u, 13) & 1
        un = u + (((15 - 127) << 23) + 0xFFF) + mant_odd
        onorm = lax.shift_right_logical(un, 13)
        return jnp.where(is_sub, osub, onorm) | s

    lo = scale_half(w & 0xFFFF)
    hi = scale_half(lax.shift_right_logical(w, 16))
    return lo, hi


def _scale_dup_body(scale, t_ref, o_ref):
    w = t_ref.bitcast(jnp.int32)[...]
    lo, hi = _scale_words(w, scale)
    o_ref[0] = lo | lax.shift_left(lo, 16)
    o_ref[1] = hi | lax.shift_left(hi, 16)


def _finish_body(w_ref, o_ref):
    o_ref.bitcast(jnp.int32)[...] = w_ref[...]


@functools.lru_cache(maxsize=None)
def _build_gather(n_rows, emb_w, half_vocab):
    rpw = n_rows // _NW
    t = rpw // _CH
    dw = 2 * emb_w
    mesh = plsc.VectorSubcoreMesh(core_axis_name="c", subcore_axis_name="s")

    @functools.partial(
        pl.kernel,
        mesh=mesh,
        out_type=jax.ShapeDtypeStruct((n_rows // 2, dw), jnp.int32),
        scratch_types=[
            pltpu.VMEM((t, _CH), jnp.int32),
            pltpu.VMEM((_NB, _CH, dw), jnp.int32),
            pltpu.VMEM((2, _CH // 2, dw), jnp.int32),
            pltpu.SemaphoreType.DMA((_NB,)),
            pltpu.SemaphoreType.DMA((2,)),
        ],
        compiler_params=pltpu.CompilerParams(use_tc_tiling_on_sc=True),
    )
    def gather(table_hbm, idx_hbm, out_hbm, idx_v, bufs, obufs, sem_g, sem_w):
        wid = lax.axis_index("s") * _NC + lax.axis_index("c")
        wbase = wid * (rpw // 2)

        pltpu.sync_copy(idx_hbm.at[pl.ds(wid * t, t)], idx_v)

        def remap(r, carry):
            for kk in range(_CH // 16):
                v = idx_v[r, pl.ds(16 * kk, 16)]
                idx_v[r, pl.ds(16 * kk, 16)] = (
                    (v & 1) * half_vocab + lax.shift_right_logical(v, 1)
                )
            return carry

        lax.fori_loop(0, t, remap, 0)

        def start_gather(j, b):
            pltpu.make_async_copy(
                table_hbm.at[idx_v.at[j]], bufs.at[b], sem_g.at[b]
            ).start()

        def wait_gather(b):
            pltpu.make_async_copy(
                table_hbm.at[pl.ds(0, _CH)], bufs.at[b], sem_g.at[b]
            ).wait()

        def start_write(j, ob):
            pltpu.make_async_copy(
                obufs.at[ob],
                out_hbm.at[pl.ds(wbase + j * (_CH // 2), _CH // 2)],
                sem_w.at[ob],
            ).start()

        def wait_write(ob):
            pltpu.make_async_copy(
                obufs.at[ob], out_hbm.at[pl.ds(wbase, _CH // 2)], sem_w.at[ob]
            ).wait()

        for b in range(_NB):
            start_gather(b, b)

        def compose(b, ob):
            @functools.partial(
                plsc.parallel_loop, 0, _CH // 2, unroll=8
            )
            def pair(i):
                for kk in range(dw // 16):
                    ga = bufs[b, 2 * i, pl.ds(16 * kk, 16)]
                    gb = bufs[b, 2 * i + 1, pl.ds(16 * kk, 16)]
                    obufs[ob, i, pl.ds(16 * kk, 16)] = (ga & 0xFFFF) | (
                        gb & ~0xFFFF
                    )

        def outer(jj, carry):
            for b in range(_NB):
                ob = b % 2
                j = jj * _NB + b
                wait_gather(b)

                @pl.when(j >= 2)
                def _():
                    wait_write(ob)

                compose(b, ob)
                start_write(j, ob)

                @pl.when(j + _NB < t)
                def _():
                    start_gather(j + _NB, b)

            return carry

        lax.fori_loop(0, t // _NB, outer, 0)
        for ob in range(2):
            wait_write(ob)

    return gather


def kernel(x, table):
    vocab, emb_dim = table.shape
    batch, seq = x.shape
    n_rows = batch * seq
    emb_w = emb_dim // 2
    scale = math.sqrt(emb_dim)

    tbits = lax.bitcast_convert_type(table, jnp.bfloat16)

    tblk = 4000
    p3 = pl.pallas_call(
        functools.partial(_scale_dup_body, scale),
        out_shape=jax.ShapeDtypeStruct((2, vocab // 2, 2 * emb_w), jnp.int32),
        grid=(vocab // tblk,),
        in_specs=[pl.BlockSpec((tblk, emb_dim), lambda i: (i, 0))],
        out_specs=pl.BlockSpec(
            (2, tblk // 2, 2 * emb_w), lambda i: (0, i, 0)
        ),
    )(tbits)

    idx = x.reshape(n_rows // _CH, _CH)
    gather = _build_gather(n_rows, emb_w, vocab // 2)
    out32 = gather(p3.reshape(vocab, 2 * emb_w), idx)

    fblk = 25600
    out = pl.pallas_call(
        _finish_body,
        out_shape=jax.ShapeDtypeStruct((n_rows, emb_dim), jnp.bfloat16),
        grid=(n_rows // fblk,),
        in_specs=[pl.BlockSpec((fblk // 2, 2 * emb_w), lambda i: (i, 0))],
        out_specs=pl.BlockSpec((fblk, emb_dim), lambda i: (i, 0)),
    )(out32)
    out = lax.bitcast_convert_type(out, table.dtype)
    return out.reshape(batch, seq, emb_dim)


# speedup vs baseline: 10.5467x; 1.2937x over previous
import functools
import math

import jax
import jax.numpy as jnp
import numpy as np
from jax import lax
from jax.experimental import pallas as pl
from jax.experimental.pallas import tpu as pltpu
from jax.experimental.pallas import tpu_sc as plsc

_NW = 32
_NC = 2
_CH = 128
_NB = 4


def _scale_words(w, scale):
    c16 = float(np.float32(np.float16(scale)))
    c_total = c16 * 2.0**112
    c_sub = c16 * 2.0**-24

    def scale_half(h):
        s = h & 0x8000
        mag = h & 0x7FFF
        f_norm = lax.bitcast_convert_type(
            lax.shift_left(mag, 13), jnp.float32
        ) * c_total
        f_sub = mag.astype(jnp.float32) * c_sub
        f = jnp.where(mag < 1024, f_sub, f_norm)
        u = lax.bitcast_convert_type(f, jnp.int32)
        is_sub = u < (113 << 23)
        vsub = f + 0.5
        osub = lax.bitcast_convert_type(vsub, jnp.int32) - (126 << 23)
        mant_odd = lax.shift_right_logical(u, 13) & 1
        un = u + (((15 - 127) << 23) + 0xFFF) + mant_odd
        onorm = lax.shift_right_logical(un, 13)
        return jnp.where(is_sub, osub, onorm) | s

    lo = scale_half(w & 0xFFFF)
    hi = scale_half(lax.shift_right_logical(w, 16))
    return lo, hi


def _scale_dup_body(scale, t_ref, o_ref):
    w = t_ref.bitcast(jnp.int32)[...]
    lo, hi = _scale_words(w, scale)
    o_ref[0] = lo | lax.shift_left(lo, 16)
    o_ref[1] = hi | lax.shift_left(hi, 16)


def _finish_body(w_ref, o_ref):
    o_ref.bitcast(jnp.int32)[...] = w_ref[...]


@functools.lru_cache(maxsize=None)
def _build_gather(n_rows, emb_w, half_vocab):
    rpw = n_rows // _NW
    t = rpw // _CH
    dw = 2 * emb_w
    mesh = plsc.VectorSubcoreMesh(core_axis_name="c", subcore_axis_name="s")

    @functools.partial(
        pl.kernel,
        mesh=mesh,
        out_type=jax.ShapeDtypeStruct((n_rows, dw), jnp.bfloat16),
        scratch_types=[
            pltpu.VMEM((t, _CH), jnp.int32),
            pltpu.VMEM((_NB, _CH, dw), jnp.int32),
            pltpu.VMEM((2, _CH, dw), jnp.bfloat16),
            pltpu.SemaphoreType.DMA((_NB,)),
            pltpu.SemaphoreType.DMA((2,)),
        ],
        compiler_params=pltpu.CompilerParams(
            use_tc_tiling_on_sc=True, needs_layout_passes=False
        ),
    )
    def gather(table_hbm, idx_hbm, out_hbm, idx_v, bufs, obufs, sem_g, sem_w):
        wid = lax.axis_index("s") * _NC + lax.axis_index("c")
        wbase = wid * rpw

        pltpu.sync_copy(idx_hbm.at[pl.ds(wid * t, t)], idx_v)

        def remap(r, carry):
            for kk in range(_CH // 16):
                v = idx_v[r, pl.ds(16 * kk, 16)]
                idx_v[r, pl.ds(16 * kk, 16)] = (
                    (v & 1) * half_vocab + lax.shift_right_logical(v, 1)
                )
            return carry

        lax.fori_loop(0, t, remap, 0)

        def start_gather(j, b):
            pltpu.make_async_copy(
                table_hbm.at[idx_v.at[j]], bufs.at[b], sem_g.at[b]
            ).start()

        def wait_gather(b):
            pltpu.make_async_copy(
                table_hbm.at[pl.ds(0, _CH)], bufs.at[b], sem_g.at[b]
            ).wait()

        def start_write(j, ob):
            pltpu.make_async_copy(
                obufs.at[ob],
                out_hbm.at[pl.ds(wbase + j * _CH, _CH)],
                sem_w.at[ob],
            ).start()

        def wait_write(ob):
            pltpu.make_async_copy(
                obufs.at[ob], out_hbm.at[pl.ds(wbase, _CH)], sem_w.at[ob]
            ).wait()

        for b in range(_NB):
            start_gather(b, b)

        def compose(b, ob):
            @functools.partial(
                plsc.parallel_loop, 0, _CH // 2, unroll=8
            )
            def pair(i):
                for kk in range(dw // 16):
                    ga = bufs[b, 2 * i, pl.ds(16 * kk, 16)]
                    gb = bufs[b, 2 * i + 1, pl.ds(16 * kk, 16)]
                    w = (ga & 0xFFFF) | (gb & ~0xFFFF)
                    obufs[
                        ob, pl.ds(2 * i, 2), pl.ds(16 * kk, 16)
                    ] = plsc.bitcast(w, jnp.bfloat16)

        def outer(jj, carry):
            for b in range(_NB):
                ob = b % 2
                j = jj * _NB + b
                wait_gather(b)

                @pl.when(j >= 2)
                def _():
                    wait_write(ob)

                compose(b, ob)
                start_write(j, ob)

                @pl.when(j + _NB < t)
                def _():
                    start_gather(j + _NB, b)

            return carry

        lax.fori_loop(0, t // _NB, outer, 0)
        for ob in range(2):
            wait_write(ob)

    return gather


def kernel(x, table):
    vocab, emb_dim = table.shape
    batch, seq = x.shape
    n_rows = batch * seq
    emb_w = emb_dim // 2
    scale = math.sqrt(emb_dim)

    tbits = lax.bitcast_convert_type(table, jnp.bfloat16)

    tblk = 4000
    p3 = pl.pallas_call(
        functools.partial(_scale_dup_body, scale),
        out_shape=jax.ShapeDtypeStruct((2, vocab // 2, 2 * emb_w), jnp.int32),
        grid=(vocab // tblk,),
        in_specs=[pl.BlockSpec((tblk, emb_dim), lambda i: (i, 0))],
        out_specs=pl.BlockSpec(
            (2, tblk // 2, 2 * emb_w), lambda i: (0, i, 0)
        ),
    )(tbits)

    idx = x.reshape(n_rows // _CH, _CH)
    gather = _build_gather(n_rows, emb_w, vocab // 2)
    out32 = gather(p3.reshape(vocab, 2 * emb_w), idx)

    out = lax.bitcast_convert_type(out32, table.dtype)
    return out.reshape(batch, seq, emb_dim)
